# baseline (speedup 1.0000x reference)
import functools

import jax
import jax.numpy as jnp
from jax import lax
from jax.experimental import pallas as pl
from jax.experimental.pallas import tpu as pltpu
from jax.experimental.pallas import tpu_sc as plsc

_N_DATA = 100000
_N_TRUNC = 25000
_E = 400000
_FEAT = 128
_F = 64
_K = 128
_NCHUNK = _E // _K
_NT = 16
_FULL = _NCHUNK // _NT
_TAIL = _NCHUNK - _FULL * _NT
_ACC_ROWS = 25600
_STRIPE = _ACC_ROWS // _NT
_NQ = 4


def _fill_zeros(zbuf):
    z = jnp.zeros((16,), jnp.float32)

    def body(i, carry):
        for t in range(_F // 16):
            zbuf[i, pl.ds(t * 16, 16)] = z
        return carry

    lax.fori_loop(0, _K, body, 0)


def _zero_stripe(zbuf, acc, tid):
    base = tid * _STRIPE

    def body(m, carry):
        pltpu.sync_copy(zbuf, acc.at[pl.ds(base + m * _K, _K)])
        return carry

    lax.fori_loop(0, _STRIPE // _K, body, 0)
    rem = _STRIPE % _K
    if rem:
        pltpu.sync_copy(zbuf.at[pl.ds(0, rem)],
                        acc.at[pl.ds(base + (_STRIPE // _K) * _K, rem)])


def _copy_stripe(acc, out_ref, tid):
    base = tid * _STRIPE

    def body(m, carry):
        pltpu.sync_copy(acc.at[pl.ds(base + m * _K, _K)],
                        out_ref.at[pl.ds(base + m * _K, _K)])
        return carry

    lax.fori_loop(0, _STRIPE // _K, body, 0)
    rem = _STRIPE % _K
    if rem:
        off = base + (_STRIPE // _K) * _K
        pltpu.sync_copy(acc.at[pl.ds(off, rem)], out_ref.at[pl.ds(off, rem)])


def _gather_scale(j, table, srcr, wr, idx_v, w_v, rows_v, row_xform):
    base = j * _K
    pltpu.sync_copy(srcr.at[pl.ds(base, _K)], idx_v)
    pltpu.sync_copy(wr.at[pl.ds(base, _K)], w_v)
    for i in range(_K // 16):
        s = idx_v[pl.ds(i * 16, 16)]
        idx_v[pl.ds(i * 16, 16)] = row_xform(s)
    pltpu.sync_copy(table.at[idx_v], rows_v)

    def sbody(g, carry):
        w16 = w_v[pl.ds(g * 16, 16)]
        for e16 in range(16):
            we = w16.at[jnp.full((16,), e16, jnp.int32)].get(
                mode="promise_in_bounds")
            e = g * 16 + e16
            for t in range(_F // 16):
                rows_v[e, pl.ds(t * 16, 16)] = (
                    rows_v[e, pl.ds(t * 16, 16)] * we)
        return carry

    lax.fori_loop(0, _K // 16, sbody, 0)


def _load_dst(j, dstr, dst_v, dst_xform):
    base = j * _K
    pltpu.sync_copy(dstr.at[pl.ds(base, _K)], dst_v)
    for i in range(_K // 16):
        d = dst_v[pl.ds(i * 16, 16)]
        dst_v[pl.ds(i * 16, 16)] = dst_xform(d)


def _down_body(xr, srcr, dstr, wr, xt_out, acc, idx_v, dst_v, rows_v, zbuf, w_v):
    c = lax.axis_index("c")
    tid = lax.axis_index("s")
    row_xform = lambda s: 2 * _N_DATA + 2 * s + c

    _fill_zeros(zbuf)
    _zero_stripe(zbuf, acc, tid)
    plsc.subcore_barrier()

    def chunk(k, carry):
        j = tid + k * _NT
        _gather_scale(j, xr, srcr, wr, idx_v, w_v, rows_v, row_xform)
        _load_dst(j, dstr, dst_v, lambda d: d)
        pltpu.sync_copy(rows_v, acc.at[dst_v], add=True)
        return carry

    lax.fori_loop(0, _FULL, chunk, 0)

    @pl.when(tid < _TAIL)
    def _tail():
        chunk(_FULL, 0)

    plsc.subcore_barrier()
    _copy_stripe(acc, xt_out.at[c], tid)


def _up_body(xtr, srcr, dstr, wr, out, acc, idx_v, dst_v, rows_v, zbuf, w_v):
    c = lax.axis_index("c")
    tid = lax.axis_index("s")
    row_xform = lambda s: c * _ACC_ROWS + s
    dummy = _N_TRUNC + 256 + tid

    _fill_zeros(zbuf)

    for q in range(_NQ):
        lo = q * _N_TRUNC

        def dst_xform(d):
            local = d - lo
            ok = (local >= 0) & (local < _N_TRUNC)
            return jnp.where(ok, local, dummy)

        _zero_stripe(zbuf, acc, tid)
        plsc.subcore_barrier()

        def chunk(k, carry):
            j = tid + k * _NT
            _gather_scale(j, xtr, srcr, wr, idx_v, w_v, rows_v, row_xform)
            _load_dst(j, dstr, dst_v, dst_xform)
            pltpu.sync_copy(rows_v, acc.at[dst_v], add=True)
            return carry

        lax.fori_loop(0, _FULL, chunk, 0)

        @pl.when(tid < _TAIL)
        def _tail():
            chunk(_FULL, 0)

        plsc.subcore_barrier()
        _copy_stripe(acc, out.at[c, q], tid)
        plsc.subcore_barrier()


@functools.lru_cache(maxsize=1)
def _build():
    mesh = plsc.VectorSubcoreMesh(core_axis_name="c", subcore_axis_name="s",
                                  num_cores=2, num_subcores=_NT)
    scratch = [
        pltpu.VMEM_SHARED((_ACC_ROWS, _F), jnp.float32),
        pltpu.VMEM((_K,), jnp.int32),
        pltpu.VMEM((_K,), jnp.int32),
        pltpu.VMEM((_K, _F), jnp.float32),
        pltpu.VMEM((_K, _F), jnp.float32),
        pltpu.VMEM((_K,), jnp.float32),
    ]
    params = pltpu.CompilerParams(use_tc_tiling_on_sc=False)
    down = pl.kernel(
        _down_body,
        out_type=jax.ShapeDtypeStruct((2, _ACC_ROWS, _F), jnp.float32),
        mesh=mesh,
        scratch_types=scratch,
        compiler_params=params,
    )
    up = pl.kernel(
        _up_body,
        out_type=jax.ShapeDtypeStruct((2, _NQ, _ACC_ROWS, _F), jnp.float32),
        mesh=mesh,
        scratch_types=scratch,
        compiler_params=params,
    )
    return down, up


def kernel(x, w_down, w_up, edge_src_down, edge_dst_down, edge_src_up,
           edge_dst_up):
    down, up = _build()
    xr = x.reshape(2 * _N_DATA * 2, _F)
    xt = down(xr, edge_src_down, edge_dst_down, w_down)
    xtr = xt.reshape(2 * _ACC_ROWS, _F)
    out4 = up(xtr, edge_src_up, edge_dst_up, w_up)
    out = out4[:, :, :_N_TRUNC, :].reshape(2, _N_DATA, _F)
    out = out.transpose(1, 0, 2).reshape(1, 1, _N_DATA, _FEAT)
    return out
